# baseline (device time: 14078 ns/iter reference)
import jax
import jax.numpy as jnp
from jax import lax
from jax.experimental import pallas as pl
from jax.experimental.pallas import tpu as pltpu

HALF = 256
CH = 32
OVF = 48
FWD = HALF - OVF

FWD_CHUNKS = [(i * CH, CH) for i in range(6)] + [(192, 16)]
OVF_CHUNKS = [(208, 32), (240, 16)]

KY = HALF // CH + len(OVF_CHUNKS)
KX = len(FWD_CHUNKS)


def kernel(x):
    m, n = x.shape

    def body(x_ref, out_ref, raw_ref, y_send, y_recv, x_send, x_recv):
        my_x = lax.axis_index("x")
        my_y = lax.axis_index("y")
        y_peer = (my_x, 1 - my_y)
        x_peer = (1 - my_x, my_y)
        off = my_x * HALF
        offr = (1 - my_x) * HALF

        barrier_sem = pltpu.get_barrier_semaphore()
        for nbr in (y_peer, x_peer):
            pl.semaphore_signal(
                barrier_sem, inc=1, device_id=nbr,
                device_id_type=pl.DeviceIdType.MESH,
            )
        pl.semaphore_wait(barrier_sem, 2)

        y_rdmas = []
        for i in range(HALF // CH):
            r = pltpu.make_async_remote_copy(
                src_ref=x_ref.at[pl.ds(off + i * CH, CH)],
                dst_ref=raw_ref.at[pl.ds(i * CH, CH)],
                send_sem=y_send.at[i],
                recv_sem=y_recv.at[i],
                device_id=y_peer,
                device_id_type=pl.DeviceIdType.MESH,
            )
            r.start()
            y_rdmas.append(r)
        for j, (rs, nr) in enumerate(OVF_CHUNKS):
            k = HALF // CH + j
            r = pltpu.make_async_remote_copy(
                src_ref=x_ref.at[pl.ds(offr + rs, nr)],
                dst_ref=raw_ref.at[pl.ds(HALF + (rs - OVF_CHUNKS[0][0]), nr)],
                send_sem=y_send.at[k],
                recv_sem=y_recv.at[k],
                device_id=y_peer,
                device_id_type=pl.DeviceIdType.MESH,
            )
            r.start()
            y_rdmas.append(r)

        x_rdmas = []
        fwd_iter = 0
        for i in range(HALF // CH):
            y_rdmas[i].wait_recv()
            rows = pl.ds(off + i * CH, CH)
            out_ref[rows, :] = x_ref[rows, :] + raw_ref[pl.ds(i * CH, CH), :]
            while fwd_iter < KX and sum(FWD_CHUNKS[fwd_iter]) <= (i + 1) * CH:
                rs, nr = FWD_CHUNKS[fwd_iter]
                frows = pl.ds(off + rs, nr)
                r = pltpu.make_async_remote_copy(
                    src_ref=out_ref.at[frows],
                    dst_ref=out_ref.at[frows],
                    send_sem=x_send.at[fwd_iter],
                    recv_sem=x_recv.at[fwd_iter],
                    device_id=x_peer,
                    device_id_type=pl.DeviceIdType.MESH,
                )
                r.start()
                x_rdmas.append(r)
                fwd_iter += 1

        for j, (rs, nr) in enumerate(OVF_CHUNKS):
            k = HALF // CH + j
            y_rdmas[k].wait_recv()
            rows = pl.ds(offr + rs, nr)
            out_ref[rows, :] = (
                x_ref[rows, :]
                + raw_ref[pl.ds(HALF + (rs - OVF_CHUNKS[0][0]), nr), :]
            )

        for i in range(KX):
            x_rdmas[i].wait_recv()
        for r in y_rdmas:
            r.wait_send()
        for r in x_rdmas:
            r.wait_send()

    return pl.pallas_call(
        body,
        out_shape=jax.ShapeDtypeStruct((m, n), x.dtype),
        in_specs=[pl.BlockSpec(memory_space=pltpu.VMEM)],
        out_specs=pl.BlockSpec(memory_space=pltpu.VMEM),
        scratch_shapes=[
            pltpu.VMEM((HALF + OVF, n), x.dtype),
            pltpu.SemaphoreType.DMA((KY,)),
            pltpu.SemaphoreType.DMA((KY,)),
            pltpu.SemaphoreType.DMA((KX,)),
            pltpu.SemaphoreType.DMA((KX,)),
        ],
        compiler_params=pltpu.CompilerParams(collective_id=0),
    )(x)


# device time: 14068 ns/iter; 1.0007x vs baseline; 1.0007x over previous
import jax
import jax.numpy as jnp
from jax import lax
from jax.experimental import pallas as pl
from jax.experimental.pallas import tpu as pltpu

HALF = 256
CH = 32
OVF = 48
FWD = HALF - OVF

FWD_CHUNKS = [(i * CH, CH) for i in range(6)] + [(192, 16)]
OVF_CHUNKS = [(208, 32), (240, 16)]

KY = HALF // CH + len(OVF_CHUNKS)
KX = len(FWD_CHUNKS)


def kernel(x):
    m, n = x.shape

    def body(x_ref, out_ref, raw_ref, y_send, y_recv, x_send, x_recv):
        my_x = lax.axis_index("x")
        my_y = lax.axis_index("y")
        y_peer = (my_x, 1 - my_y)
        x_peer = (1 - my_x, my_y)
        off = my_x * HALF
        offr = (1 - my_x) * HALF

        barrier_sem = pltpu.get_barrier_semaphore()
        for nbr in (y_peer, x_peer):
            pl.semaphore_signal(
                barrier_sem, inc=1, device_id=nbr,
                device_id_type=pl.DeviceIdType.MESH,
            )
        pl.semaphore_wait(barrier_sem, 2)

        y_rdmas = [None] * KY
        nc = HALF // CH
        issue_order = list(range(nc - 1)) + list(range(nc, KY)) + [nc - 1]
        for i in range(nc):
            y_rdmas[i] = pltpu.make_async_remote_copy(
                src_ref=x_ref.at[pl.ds(off + i * CH, CH)],
                dst_ref=raw_ref.at[pl.ds(i * CH, CH)],
                send_sem=y_send.at[i],
                recv_sem=y_recv.at[i],
                device_id=y_peer,
                device_id_type=pl.DeviceIdType.MESH,
            )
        for j, (rs, nr) in enumerate(OVF_CHUNKS):
            k = nc + j
            y_rdmas[k] = pltpu.make_async_remote_copy(
                src_ref=x_ref.at[pl.ds(offr + rs, nr)],
                dst_ref=raw_ref.at[pl.ds(HALF + (rs - OVF_CHUNKS[0][0]), nr)],
                send_sem=y_send.at[k],
                recv_sem=y_recv.at[k],
                device_id=y_peer,
                device_id_type=pl.DeviceIdType.MESH,
            )
        for k in issue_order:
            y_rdmas[k].start()

        x_rdmas = []
        fwd_iter = 0
        for i in range(HALF // CH):
            y_rdmas[i].wait_recv()
            rows = pl.ds(off + i * CH, CH)
            out_ref[rows, :] = x_ref[rows, :] + raw_ref[pl.ds(i * CH, CH), :]
            while fwd_iter < KX and sum(FWD_CHUNKS[fwd_iter]) <= (i + 1) * CH:
                rs, nr = FWD_CHUNKS[fwd_iter]
                frows = pl.ds(off + rs, nr)
                r = pltpu.make_async_remote_copy(
                    src_ref=out_ref.at[frows],
                    dst_ref=out_ref.at[frows],
                    send_sem=x_send.at[fwd_iter],
                    recv_sem=x_recv.at[fwd_iter],
                    device_id=x_peer,
                    device_id_type=pl.DeviceIdType.MESH,
                )
                r.start()
                x_rdmas.append(r)
                fwd_iter += 1

        for j, (rs, nr) in enumerate(OVF_CHUNKS):
            k = HALF // CH + j
            y_rdmas[k].wait_recv()
            rows = pl.ds(offr + rs, nr)
            out_ref[rows, :] = (
                x_ref[rows, :]
                + raw_ref[pl.ds(HALF + (rs - OVF_CHUNKS[0][0]), nr), :]
            )

        for i in range(KX):
            x_rdmas[i].wait_recv()
        for r in y_rdmas:
            r.wait_send()
        for r in x_rdmas:
            r.wait_send()

    return pl.pallas_call(
        body,
        out_shape=jax.ShapeDtypeStruct((m, n), x.dtype),
        in_specs=[pl.BlockSpec(memory_space=pltpu.VMEM)],
        out_specs=pl.BlockSpec(memory_space=pltpu.VMEM),
        scratch_shapes=[
            pltpu.VMEM((HALF + OVF, n), x.dtype),
            pltpu.SemaphoreType.DMA((KY,)),
            pltpu.SemaphoreType.DMA((KY,)),
            pltpu.SemaphoreType.DMA((KX,)),
            pltpu.SemaphoreType.DMA((KX,)),
        ],
        compiler_params=pltpu.CompilerParams(collective_id=0),
    )(x)
